# Initial kernel scaffold; baseline (speedup 1.0000x reference)
import jax
import jax.numpy as jnp
from jax.experimental import pallas as pl


def kernel(x, edge_index, W1, b1, beta, W2, b2):
    raise NotImplementedError("write your pallas kernel here")


# speedup vs baseline: 1.0985x; 1.0985x over previous
import jax
import jax.numpy as jnp
from jax import lax
from jax.experimental import pallas as pl

N = 50000
E = 800000
F_IN = 100
HID = 32
C = 2


def _tc_prep_body(x_ref, w1_ref, b1_ref, beta_ref, h_ref, hn_ref, inv_ref):
    h = jnp.maximum(
        jnp.dot(x_ref[...], w1_ref[...], preferred_element_type=jnp.float32)
        + b1_ref[...], 0.0)
    ss = jnp.sum(h * h, axis=1, keepdims=True)
    inv = lax.rsqrt(jnp.maximum(ss, 1e-24))
    h_ref[...] = h
    hn_ref[...] = h * inv
    inv_ref[...] = inv * beta_ref[0, 0]


def _tc_prep(x, W1, b1, beta):
    blk = 400
    return pl.pallas_call(
        _tc_prep_body,
        grid=(N // blk,),
        in_specs=[
            pl.BlockSpec((blk, F_IN), lambda i: (i, 0)),
            pl.BlockSpec((F_IN, HID), lambda i: (0, 0)),
            pl.BlockSpec((1, HID), lambda i: (0, 0)),
            pl.BlockSpec((1, 1), lambda i: (0, 0)),
        ],
        out_specs=[
            pl.BlockSpec((blk, HID), lambda i: (i, 0)),
            pl.BlockSpec((blk, HID), lambda i: (i, 0)),
            pl.BlockSpec((blk, 1), lambda i: (i, 0)),
        ],
        out_shape=[
            jax.ShapeDtypeStruct((N, HID), jnp.float32),
            jax.ShapeDtypeStruct((N, HID), jnp.float32),
            jax.ShapeDtypeStruct((N, 1), jnp.float32),
        ],
    )(x, W1, b1.reshape(1, HID), beta.reshape(1, 1))


def _tc_finish_body(num_ref, den_ref, w2_ref, b2_ref, out_ref):
    out = num_ref[...] / (den_ref[...] + 1e-16)
    logits = jnp.dot(out, w2_ref[...], preferred_element_type=jnp.float32) + b2_ref[...]
    m = jnp.max(logits, axis=1, keepdims=True)
    out_ref[...] = logits - m - jnp.log(
        jnp.sum(jnp.exp(logits - m), axis=1, keepdims=True))


def _tc_finish(num, den, W2, b2):
    blk = 400
    return pl.pallas_call(
        _tc_finish_body,
        grid=(N // blk,),
        in_specs=[
            pl.BlockSpec((blk, HID), lambda i: (i, 0)),
            pl.BlockSpec((blk, 1), lambda i: (i, 0)),
            pl.BlockSpec((HID, C), lambda i: (0, 0)),
            pl.BlockSpec((1, C), lambda i: (0, 0)),
        ],
        out_specs=pl.BlockSpec((blk, C), lambda i: (i, 0)),
        out_shape=jax.ShapeDtypeStruct((N, C), jnp.float32),
    )(num, den, W2, b2.reshape(1, C))


def kernel(x, edge_index, W1, b1, beta, W2, b2):
    h, hn, invb = _tc_prep(x, W1, b1, beta)
    src = edge_index[0]
    dst = edge_index[1]
    alpha = jnp.sum(h[src] * hn[dst], axis=-1) * invb[src, 0]
    w = jnp.exp(alpha)
    den = jax.ops.segment_sum(w, dst, num_segments=N)
    num = jax.ops.segment_sum(w[:, None] * h[src], dst, num_segments=N)
    return _tc_finish(num, den[:, None], W2, b2)


# speedup vs baseline: 1.1036x; 1.0047x over previous
import jax
import jax.numpy as jnp
from jax import lax
from jax.experimental import pallas as pl

N = 50000
E = 800000
F_IN = 100
HID = 32
C = 2


def _tc_prep_body(x_ref, w1_ref, b1_ref, beta_ref, h_ref, hn_ref, inv_ref):
    h = jnp.maximum(
        jnp.dot(x_ref[...], w1_ref[...], preferred_element_type=jnp.float32)
        + b1_ref[...], 0.0)
    ss = jnp.sum(h * h, axis=1, keepdims=True)
    inv = lax.rsqrt(jnp.maximum(ss, 1e-24))
    h_ref[...] = h
    hn_ref[...] = h * inv
    inv_ref[...] = inv * beta_ref[0, 0]


def _tc_prep(x, W1, b1, beta):
    blk = 400
    return pl.pallas_call(
        _tc_prep_body,
        grid=(N // blk,),
        in_specs=[
            pl.BlockSpec((blk, F_IN), lambda i: (i, 0)),
            pl.BlockSpec((F_IN, HID), lambda i: (0, 0)),
            pl.BlockSpec((1, HID), lambda i: (0, 0)),
            pl.BlockSpec((1, 1), lambda i: (0, 0)),
        ],
        out_specs=[
            pl.BlockSpec((blk, HID), lambda i: (i, 0)),
            pl.BlockSpec((blk, HID), lambda i: (i, 0)),
            pl.BlockSpec((blk, 1), lambda i: (i, 0)),
        ],
        out_shape=[
            jax.ShapeDtypeStruct((N, HID), jnp.float32),
            jax.ShapeDtypeStruct((N, HID), jnp.float32),
            jax.ShapeDtypeStruct((N, 1), jnp.float32),
        ],
    )(x, W1, b1.reshape(1, HID), beta.reshape(1, 1))


def _tc_finish_body(num_ref, den_ref, w2_ref, b2_ref, out_ref):
    out = num_ref[...] / (den_ref[...] + 1e-16)
    logits = jnp.dot(out, w2_ref[...], preferred_element_type=jnp.float32) + b2_ref[...]
    m = jnp.max(logits, axis=1, keepdims=True)
    out_ref[...] = logits - m - jnp.log(
        jnp.sum(jnp.exp(logits - m), axis=1, keepdims=True))


def _tc_finish(num, den, W2, b2):
    blk = 400
    return pl.pallas_call(
        _tc_finish_body,
        grid=(N // blk,),
        in_specs=[
            pl.BlockSpec((blk, HID), lambda i: (i, 0)),
            pl.BlockSpec((blk, 1), lambda i: (i, 0)),
            pl.BlockSpec((HID, C), lambda i: (0, 0)),
            pl.BlockSpec((1, C), lambda i: (0, 0)),
        ],
        out_specs=pl.BlockSpec((blk, C), lambda i: (i, 0)),
        out_shape=jax.ShapeDtypeStruct((N, C), jnp.float32),
    )(num, den, W2, b2.reshape(1, C))


def kernel(x, edge_index, W1, b1, beta, W2, b2):
    h, hn, invb = _tc_prep(x, W1, b1, beta)
    src = edge_index[0]
    dst = edge_index[1]
    alpha = jnp.sum(h[src] * hn[dst], axis=-1) * invb[src, 0]
    w = jnp.exp(alpha)
    den = jax.ops.segment_sum(w, dst, num_segments=N)
    num = jax.ops.segment_sum(w[:, None] * h[src], dst, num_segments=N)
    out = num / (den[:, None] + 1e-16)
    logits = out @ W2 + b2
    return jax.nn.log_softmax(logits, axis=1)


# speedup vs baseline: 1.6473x; 1.4926x over previous
import jax
import jax.numpy as jnp
from jax import lax
from jax.experimental import pallas as pl

N = 50000
E = 800000
F_IN = 100
HID = 32
C = 2
SW = 48


def _tc_prep_body(x_ref, w1_ref, b1_ref, beta_ref, src_ref, dst_ref):
    h = jnp.maximum(
        jnp.dot(x_ref[...], w1_ref[...], preferred_element_type=jnp.float32)
        + b1_ref[...], 0.0)
    ss = jnp.sum(h * h, axis=1, keepdims=True)
    inv = lax.rsqrt(jnp.maximum(ss, 1e-24))
    dst_ref[...] = h * inv
    rows = h.shape[0]
    src_ref[...] = jnp.concatenate(
        [h, inv * beta_ref[0, 0], jnp.zeros((rows, SW - HID - 1), jnp.float32)],
        axis=1)


def _tc_prep(x, W1, b1, beta):
    blk = 400
    return pl.pallas_call(
        _tc_prep_body,
        grid=(N // blk,),
        in_specs=[
            pl.BlockSpec((blk, F_IN), lambda i: (i, 0)),
            pl.BlockSpec((F_IN, HID), lambda i: (0, 0)),
            pl.BlockSpec((1, HID), lambda i: (0, 0)),
            pl.BlockSpec((1, 1), lambda i: (0, 0)),
        ],
        out_specs=[
            pl.BlockSpec((blk, SW), lambda i: (i, 0)),
            pl.BlockSpec((blk, HID), lambda i: (i, 0)),
        ],
        out_shape=[
            jax.ShapeDtypeStruct((N, SW), jnp.float32),
            jax.ShapeDtypeStruct((N, HID), jnp.float32),
        ],
    )(x, W1, b1.reshape(1, HID), beta.reshape(1, 1))


def kernel(x, edge_index, W1, b1, beta, W2, b2):
    srcT, dstT = _tc_prep(x, W1, b1, beta)
    h = srcT[:, :HID]
    invb = srcT[:, HID]
    hn = dstT
    src = edge_index[0]
    dst = edge_index[1]
    alpha = jnp.sum(h[src] * hn[dst], axis=-1) * invb[src]
    w = jnp.exp(alpha)
    denom = jax.ops.segment_sum(w, dst, num_segments=N)
    num = jax.ops.segment_sum(w[:, None] * h[src], dst, num_segments=N)
    out = num / (denom[:, None] + 1e-16)
    logits = out @ W2 + b2
    return jax.nn.log_softmax(logits, axis=1)
